# initial kernel scaffold (unmeasured)
import jax
import jax.numpy as jnp
from jax import lax
from jax.experimental import pallas as pl
from jax.experimental.pallas import tpu as pltpu

N_DEV = 8


def kernel(x, Win0, Wout0, Win1, Wout1, Win2, Wout2):
    m_per, d = x.shape
    M = N_DEV * m_per
    n_hops = (N_DEV - 1) * 7

    def body(x_ref, win0_ref, wout0_ref, win1_ref, wout1_ref, win2_ref,
             wout2_ref, out_ref, xfull, partial, rs_recv, rs_acc,
             send_sems, recv_sems):
        my = lax.axis_index("i")
        right = lax.rem(my + 1, N_DEV)

        hops = [0]

        def ring_send(src, dst):
            h = hops[0]
            hops[0] += 1
            rdma = pltpu.make_async_remote_copy(
                src_ref=src,
                dst_ref=dst,
                send_sem=send_sems.at[h],
                recv_sem=recv_sems.at[h],
                device_id=(right,),
                device_id_type=pl.DeviceIdType.MESH,
            )
            rdma.start()
            rdma.wait()

        def chunk(ref, c):
            return ref.at[pl.ds(c * m_per, m_per), :]

        xfull[pl.ds(my * m_per, m_per), :] = x_ref[:, :]
        for h in range(N_DEV - 1):
            c = lax.rem(my - h + N_DEV, N_DEV)
            ring_send(chunk(xfull, c), chunk(xfull, c))

        for win_ref, wout_ref in (
            (win0_ref, wout0_ref),
            (win1_ref, wout1_ref),
            (win2_ref, wout2_ref),
        ):
            h_act = jnp.maximum(
                jnp.dot(xfull[:, :], win_ref[:, :],
                        preferred_element_type=jnp.float32),
                0.0,
            )
            partial[:, :] = jnp.dot(h_act, wout_ref[:, :],
                                    preferred_element_type=jnp.float32)

            for s in range(N_DEV - 1):
                if s == 0:
                    src = chunk(partial, my)
                else:
                    src = rs_acc.at[s - 1]
                ring_send(src, rs_recv.at[s])
                c_recv = lax.rem(my - s - 1 + N_DEV, N_DEV)
                rs_acc[s, :, :] = (
                    rs_recv[s, :, :]
                    + partial[pl.ds(c_recv * m_per, m_per), :]
                )

            o = lax.rem(my + 1, N_DEV)
            xfull[pl.ds(o * m_per, m_per), :] = rs_acc[N_DEV - 2, :, :]
            for h in range(N_DEV - 1):
                c = lax.rem(o - h + N_DEV, N_DEV)
                ring_send(chunk(xfull, c), chunk(xfull, c))

        out_ref[:, :] = xfull[:, :]

    return pl.pallas_call(
        body,
        out_shape=jax.ShapeDtypeStruct((M, d), jnp.float32),
        in_specs=[pl.BlockSpec(memory_space=pltpu.VMEM)] * 7,
        out_specs=pl.BlockSpec(memory_space=pltpu.VMEM),
        scratch_shapes=[
            pltpu.VMEM((M, d), jnp.float32),
            pltpu.VMEM((M, d), jnp.float32),
            pltpu.VMEM((N_DEV - 1, m_per, d), jnp.float32),
            pltpu.VMEM((N_DEV - 1, m_per, d), jnp.float32),
            pltpu.SemaphoreType.DMA((n_hops,)),
            pltpu.SemaphoreType.DMA((n_hops,)),
        ],
        compiler_params=pltpu.CompilerParams(collective_id=0),
    )(x, Win0, Wout0, Win1, Wout1, Win2, Wout2)


# baseline (device time: 247357 ns/iter reference)
import jax
import jax.numpy as jnp
from jax import lax
from jax.experimental import pallas as pl
from jax.experimental.pallas import tpu as pltpu

N_DEV = 8


def kernel(x, Win0, Wout0, Win1, Wout1, Win2, Wout2):
    m_per, d = x.shape
    M = N_DEV * m_per
    n_hops = (N_DEV - 1) * 7

    def body(x_ref, win0_ref, wout0_ref, win1_ref, wout1_ref, win2_ref,
             wout2_ref, out_ref, xfull, partial, rs_recv, rs_acc,
             send_sems, recv_sems):
        my = lax.axis_index("i")
        right = lax.rem(my + 1, N_DEV)

        hops = [0]

        def ring_send(src, dst):
            h = hops[0]
            hops[0] += 1
            rdma = pltpu.make_async_remote_copy(
                src_ref=src,
                dst_ref=dst,
                send_sem=send_sems.at[h],
                recv_sem=recv_sems.at[h],
                device_id=(right,),
                device_id_type=pl.DeviceIdType.MESH,
            )
            rdma.start()
            rdma.wait()

        def chunk(ref, c):
            return ref.at[pl.ds(c * m_per, m_per), :]

        xfull[pl.ds(my * m_per, m_per), :] = x_ref[:, :]
        for h in range(N_DEV - 1):
            c = lax.rem(my - h + N_DEV, N_DEV)
            ring_send(chunk(xfull, c), chunk(xfull, c))

        for win_ref, wout_ref in (
            (win0_ref, wout0_ref),
            (win1_ref, wout1_ref),
            (win2_ref, wout2_ref),
        ):
            h_act = jnp.maximum(
                jnp.dot(xfull[:, :], win_ref[:, :],
                        preferred_element_type=jnp.float32),
                0.0,
            )
            partial[:, :] = jnp.dot(h_act, wout_ref[:, :],
                                    preferred_element_type=jnp.float32)

            for s in range(N_DEV - 1):
                if s == 0:
                    src = chunk(partial, my)
                else:
                    src = rs_acc.at[s - 1]
                ring_send(src, rs_recv.at[s])
                c_recv = lax.rem(my - s - 1 + N_DEV, N_DEV)
                rs_acc[s, :, :] = (
                    rs_recv[s, :, :]
                    + partial[pl.ds(c_recv * m_per, m_per), :]
                )

            o = lax.rem(my + 1, N_DEV)
            xfull[pl.ds(o * m_per, m_per), :] = rs_acc[N_DEV - 2, :, :]
            for h in range(N_DEV - 1):
                c = lax.rem(o - h + N_DEV, N_DEV)
                ring_send(chunk(xfull, c), chunk(xfull, c))

        out_ref[:, :] = xfull[:, :]

    return pl.pallas_call(
        body,
        out_shape=jax.ShapeDtypeStruct((M, d), jnp.float32),
        in_specs=[pl.BlockSpec(memory_space=pltpu.VMEM)] * 7,
        out_specs=pl.BlockSpec(memory_space=pltpu.VMEM),
        scratch_shapes=[
            pltpu.VMEM((M, d), jnp.float32),
            pltpu.VMEM((M, d), jnp.float32),
            pltpu.VMEM((N_DEV - 1, m_per, d), jnp.float32),
            pltpu.VMEM((N_DEV - 1, m_per, d), jnp.float32),
            pltpu.SemaphoreType.DMA((n_hops,)),
            pltpu.SemaphoreType.DMA((n_hops,)),
        ],
    )(x, Win0, Wout0, Win1, Wout1, Win2, Wout2)


# device time: 190464 ns/iter; 1.2987x vs baseline; 1.2987x over previous
import jax
import jax.numpy as jnp
from jax import lax
from jax.experimental import pallas as pl
from jax.experimental.pallas import tpu as pltpu

N_DEV = 8


def kernel(x, Win0, Wout0, Win1, Wout1, Win2, Wout2):
    m_per, d = x.shape
    M = N_DEV * m_per
    n_ex = 3 + 3 * 6

    def body(x_ref, win0_ref, wout0_ref, win1_ref, wout1_ref, win2_ref,
             wout2_ref, out_ref, xfull, partial, rs_recv,
             send_sems, recv_sems):
        my = lax.axis_index("i")
        p1 = jnp.bitwise_xor(my, 1)
        p2 = jnp.bitwise_xor(my, 3)
        p3 = jnp.bitwise_xor(my, 4)
        b0 = jnp.bitwise_and(my, 1)
        b1 = jnp.bitwise_and(my >> 1, 1)
        b2 = jnp.bitwise_and(my >> 2, 1)

        barrier_sem = pltpu.get_barrier_semaphore()
        for p in (p1, p2, p3):
            pl.semaphore_signal(
                barrier_sem, inc=1,
                device_id=(p,), device_id_type=pl.DeviceIdType.MESH,
            )
        pl.semaphore_wait(barrier_sem, 3)

        ex = [0]

        def exchange(src, dst, partner):
            k = ex[0]
            ex[0] += 1
            rdma = pltpu.make_async_remote_copy(
                src_ref=src,
                dst_ref=dst,
                send_sem=send_sems.at[k],
                recv_sem=recv_sems.at[k],
                device_id=(partner,),
                device_id_type=pl.DeviceIdType.MESH,
            )
            rdma.start()
            rdma.wait()

        def rows(ref, start, length):
            return ref.at[pl.ds(start, length), :]

        xfull[pl.ds(my * m_per, m_per), :] = x_ref[:, :]
        a0 = my * m_per
        exchange(rows(xfull, a0, m_per), rows(xfull, a0, m_per), p1)
        a1 = (my - b0) * m_per
        exchange(rows(xfull, a1, 2 * m_per), rows(xfull, a1, 2 * m_per), p2)
        a2 = (my - jnp.bitwise_and(my, 3)) * m_per
        exchange(rows(xfull, a2, 4 * m_per), rows(xfull, a2, 4 * m_per), p3)

        for win_ref, wout_ref in (
            (win0_ref, wout0_ref),
            (win1_ref, wout1_ref),
            (win2_ref, wout2_ref),
        ):
            h_act = jnp.maximum(
                jnp.dot(xfull[:, :], win_ref[:, :],
                        preferred_element_type=jnp.float32),
                0.0,
            )
            partial[:, :] = jnp.dot(h_act, wout_ref[:, :],
                                    preferred_element_type=jnp.float32)

            H = b2 * (4 * m_per)
            exchange(rows(partial, (1 - b2) * 4 * m_per, 4 * m_per),
                     rs_recv.at[0, pl.ds(0, 4 * m_per), :], p3)
            partial[pl.ds(H, 4 * m_per), :] = (
                partial[pl.ds(H, 4 * m_per), :]
                + rs_recv[0, 0:4 * m_per, :]
            )
            Q = H + b1 * (2 * m_per)
            exchange(rows(partial, H + (1 - b1) * 2 * m_per, 2 * m_per),
                     rs_recv.at[1, pl.ds(0, 2 * m_per), :], p2)
            partial[pl.ds(Q, 2 * m_per), :] = (
                partial[pl.ds(Q, 2 * m_per), :]
                + rs_recv[1, 0:2 * m_per, :]
            )
            E = Q + b0 * m_per
            exchange(rows(partial, Q + (1 - b0) * m_per, m_per),
                     rs_recv.at[2, pl.ds(0, m_per), :], p1)
            xfull[pl.ds(E, m_per), :] = (
                partial[pl.ds(E, m_per), :]
                + rs_recv[2, 0:m_per, :]
            )

            exchange(rows(xfull, E, m_per), rows(xfull, E, m_per), p1)
            exchange(rows(xfull, Q, 2 * m_per), rows(xfull, Q, 2 * m_per), p2)
            exchange(rows(xfull, H, 4 * m_per), rows(xfull, H, 4 * m_per), p3)

        out_ref[:, :] = xfull[:, :]

    return pl.pallas_call(
        body,
        out_shape=jax.ShapeDtypeStruct((M, d), jnp.float32),
        in_specs=[pl.BlockSpec(memory_space=pltpu.VMEM)] * 7,
        out_specs=pl.BlockSpec(memory_space=pltpu.VMEM),
        scratch_shapes=[
            pltpu.VMEM((M, d), jnp.float32),
            pltpu.VMEM((M, d), jnp.float32),
            pltpu.VMEM((3, 4 * m_per, d), jnp.float32),
            pltpu.SemaphoreType.DMA((n_ex,)),
            pltpu.SemaphoreType.DMA((n_ex,)),
        ],
        compiler_params=pltpu.CompilerParams(collective_id=0),
    )(x, Win0, Wout0, Win1, Wout1, Win2, Wout2)


# device time: 122547 ns/iter; 2.0185x vs baseline; 1.5542x over previous
import jax
import jax.numpy as jnp
from jax import lax
from jax.experimental import pallas as pl
from jax.experimental.pallas import tpu as pltpu

N_DEV = 8


def kernel(x, Win0, Wout0, Win1, Wout1, Win2, Wout2):
    m_per, d = x.shape
    M = N_DEV * m_per
    n_ex = 3 + 3 * 6

    def body(x_ref, win0_ref, wout0_ref, win1_ref, wout1_ref, win2_ref,
             wout2_ref, out_ref, xbf, partial, pbf, rs_recv,
             send_sems, recv_sems):
        my = lax.axis_index("i")
        p1 = jnp.bitwise_xor(my, 1)
        p2 = jnp.bitwise_xor(my, 3)
        p3 = jnp.bitwise_xor(my, 4)
        b0 = jnp.bitwise_and(my, 1)
        b1 = jnp.bitwise_and(my >> 1, 1)
        b2 = jnp.bitwise_and(my >> 2, 1)

        barrier_sem = pltpu.get_barrier_semaphore()
        for p in (p1, p2, p3):
            pl.semaphore_signal(
                barrier_sem, inc=1,
                device_id=(p,), device_id_type=pl.DeviceIdType.MESH,
            )
        pl.semaphore_wait(barrier_sem, 3)

        ex = [0]

        def exchange(src, dst, partner):
            k = ex[0]
            ex[0] += 1
            rdma = pltpu.make_async_remote_copy(
                src_ref=src,
                dst_ref=dst,
                send_sem=send_sems.at[k],
                recv_sem=recv_sems.at[k],
                device_id=(partner,),
                device_id_type=pl.DeviceIdType.MESH,
            )
            rdma.start()
            rdma.wait()

        def rows(ref, start, length):
            return ref.at[pl.ds(start, length), :]

        xbf[pl.ds(my * m_per, m_per), :] = x_ref[:, :].astype(jnp.bfloat16)
        a0 = my * m_per
        exchange(rows(xbf, a0, m_per), rows(xbf, a0, m_per), p1)
        a1 = (my - b0) * m_per
        exchange(rows(xbf, a1, 2 * m_per), rows(xbf, a1, 2 * m_per), p2)
        a2 = (my - jnp.bitwise_and(my, 3)) * m_per
        exchange(rows(xbf, a2, 4 * m_per), rows(xbf, a2, 4 * m_per), p3)

        for win_ref, wout_ref in (
            (win0_ref, wout0_ref),
            (win1_ref, wout1_ref),
            (win2_ref, wout2_ref),
        ):
            h_act = jnp.maximum(
                jnp.dot(xbf[:, :], win_ref[:, :].astype(jnp.bfloat16),
                        preferred_element_type=jnp.float32),
                0.0,
            )
            partial[:, :] = jnp.dot(
                h_act.astype(jnp.bfloat16),
                wout_ref[:, :].astype(jnp.bfloat16),
                preferred_element_type=jnp.float32,
            )

            H = b2 * (4 * m_per)
            S = (1 - b2) * (4 * m_per)
            pbf[pl.ds(S, 4 * m_per), :] = (
                partial[pl.ds(S, 4 * m_per), :].astype(jnp.bfloat16))
            exchange(rows(pbf, S, 4 * m_per),
                     rs_recv.at[0, pl.ds(0, 4 * m_per), :], p3)
            partial[pl.ds(H, 4 * m_per), :] = (
                partial[pl.ds(H, 4 * m_per), :]
                + rs_recv[0, 0:4 * m_per, :].astype(jnp.float32)
            )
            Q = H + b1 * (2 * m_per)
            S = H + (1 - b1) * (2 * m_per)
            pbf[pl.ds(S, 2 * m_per), :] = (
                partial[pl.ds(S, 2 * m_per), :].astype(jnp.bfloat16))
            exchange(rows(pbf, S, 2 * m_per),
                     rs_recv.at[1, pl.ds(0, 2 * m_per), :], p2)
            partial[pl.ds(Q, 2 * m_per), :] = (
                partial[pl.ds(Q, 2 * m_per), :]
                + rs_recv[1, 0:2 * m_per, :].astype(jnp.float32)
            )
            E = Q + b0 * m_per
            S = Q + (1 - b0) * m_per
            pbf[pl.ds(S, m_per), :] = (
                partial[pl.ds(S, m_per), :].astype(jnp.bfloat16))
            exchange(rows(pbf, S, m_per),
                     rs_recv.at[2, pl.ds(0, m_per), :], p1)
            xbf[pl.ds(E, m_per), :] = (
                partial[pl.ds(E, m_per), :]
                + rs_recv[2, 0:m_per, :].astype(jnp.float32)
            ).astype(jnp.bfloat16)

            exchange(rows(xbf, E, m_per), rows(xbf, E, m_per), p1)
            exchange(rows(xbf, Q, 2 * m_per), rows(xbf, Q, 2 * m_per), p2)
            exchange(rows(xbf, H, 4 * m_per), rows(xbf, H, 4 * m_per), p3)

        out_ref[:, :] = xbf[:, :].astype(jnp.float32)

    return pl.pallas_call(
        body,
        out_shape=jax.ShapeDtypeStruct((M, d), jnp.float32),
        in_specs=[pl.BlockSpec(memory_space=pltpu.VMEM)] * 7,
        out_specs=pl.BlockSpec(memory_space=pltpu.VMEM),
        scratch_shapes=[
            pltpu.VMEM((M, d), jnp.bfloat16),
            pltpu.VMEM((M, d), jnp.float32),
            pltpu.VMEM((M, d), jnp.bfloat16),
            pltpu.VMEM((3, 4 * m_per, d), jnp.bfloat16),
            pltpu.SemaphoreType.DMA((n_ex,)),
            pltpu.SemaphoreType.DMA((n_ex,)),
        ],
        compiler_params=pltpu.CompilerParams(collective_id=0),
    )(x, Win0, Wout0, Win1, Wout1, Win2, Wout2)


# device time: 95574 ns/iter; 2.5881x vs baseline; 1.2822x over previous
import jax
import jax.numpy as jnp
from jax import lax
from jax.experimental import pallas as pl
from jax.experimental.pallas import tpu as pltpu

N_DEV = 8


def kernel(x, Win0, Wout0, Win1, Wout1, Win2, Wout2):
    m_per, d = x.shape
    M = N_DEV * m_per
    cw = d // 2
    n_ex = 3 + 3 * 12

    def body(x_ref, win0_ref, wout0_ref, win1_ref, wout1_ref, win2_ref,
             wout2_ref, out_ref, xbf, partial, pbf, rs_recv,
             send_sems, recv_sems):
        my = lax.axis_index("i")
        p1 = jnp.bitwise_xor(my, 1)
        p2 = jnp.bitwise_xor(my, 3)
        p3 = jnp.bitwise_xor(my, 4)
        b0 = jnp.bitwise_and(my, 1)
        b1 = jnp.bitwise_and(my >> 1, 1)
        b2 = jnp.bitwise_and(my >> 2, 1)

        barrier_sem = pltpu.get_barrier_semaphore()
        for p in (p1, p2, p3):
            pl.semaphore_signal(
                barrier_sem, inc=1,
                device_id=(p,), device_id_type=pl.DeviceIdType.MESH,
            )
        pl.semaphore_wait(barrier_sem, 3)

        ex = [0]

        def start_ex(src, dst, partner):
            k = ex[0]
            ex[0] += 1
            rdma = pltpu.make_async_remote_copy(
                src_ref=src,
                dst_ref=dst,
                send_sem=send_sems.at[k],
                recv_sem=recv_sems.at[k],
                device_id=(partner,),
                device_id_type=pl.DeviceIdType.MESH,
            )
            rdma.start()
            return rdma

        def exchange(src, dst, partner):
            start_ex(src, dst, partner).wait()

        xbf[pl.ds(my * m_per, m_per), :] = x_ref[:, :].astype(jnp.bfloat16)
        a0 = my * m_per
        exchange(xbf.at[pl.ds(a0, m_per), :],
                 xbf.at[pl.ds(a0, m_per), :], p1)
        a1 = (my - b0) * m_per
        exchange(xbf.at[pl.ds(a1, 2 * m_per), :],
                 xbf.at[pl.ds(a1, 2 * m_per), :], p2)
        a2 = (my - jnp.bitwise_and(my, 3)) * m_per
        exchange(xbf.at[pl.ds(a2, 4 * m_per), :],
                 xbf.at[pl.ds(a2, 4 * m_per), :], p3)

        HA = b2 * (4 * m_per)
        QA = HA + b1 * (2 * m_per)
        EA = QA + b0 * m_per
        SA = (
            (1 - b2) * (4 * m_per),
            HA + (1 - b1) * (2 * m_per),
            QA + (1 - b0) * m_per,
        )
        HB = b1 * (4 * m_per)
        QB = HB + b2 * (2 * m_per)
        EB = QB + b0 * m_per
        SB = (
            (1 - b1) * (4 * m_per),
            HB + (1 - b2) * (2 * m_per),
            QB + (1 - b0) * m_per,
        )
        chains = (
            dict(c0=0, H=HA, Q=QA, E=EA, S=SA, rs_p=(p3, p2, p1),
                 ag_p=(p1, p2, p3), idx=0),
            dict(c0=cw, H=HB, Q=QB, E=EB, S=SB, rs_p=(p2, p3, p1),
                 ag_p=(p1, p3, p2), idx=1),
        )
        sizes = (4 * m_per, 2 * m_per, m_per)
        keeps = lambda ch: (ch["H"], ch["Q"], ch["E"])
        ag_offs = lambda ch: (ch["E"], ch["Q"], ch["H"])
        ag_lens = (m_per, 2 * m_per, 4 * m_per)

        def blk(ref, r, L, c0):
            return ref.at[pl.ds(r, L), pl.ds(c0, cw)]

        def cast_send(ch, st):
            r, L, c0 = ch["S"][st], sizes[st], ch["c0"]
            pbf[pl.ds(r, L), pl.ds(c0, cw)] = (
                partial[pl.ds(r, L), pl.ds(c0, cw)].astype(jnp.bfloat16))

        def rs_start(ch, st):
            r, L = ch["S"][st], sizes[st]
            return start_ex(
                blk(pbf, r, L, ch["c0"]),
                rs_recv.at[ch["idx"], st, pl.ds(0, L), :],
                ch["rs_p"][st],
            )

        def rs_finish(ch, st):
            k, L, c0 = keeps(ch)[st], sizes[st], ch["c0"]
            acc = (
                partial[pl.ds(k, L), pl.ds(c0, cw)]
                + rs_recv[ch["idx"], st, 0:L, :].astype(jnp.float32)
            )
            if st == 2:
                xbf[pl.ds(k, L), pl.ds(c0, cw)] = acc.astype(jnp.bfloat16)
            else:
                partial[pl.ds(k, L), pl.ds(c0, cw)] = acc

        def ag_start(ch, st):
            r, L = ag_offs(ch)[st], ag_lens[st]
            return start_ex(blk(xbf, r, L, ch["c0"]),
                            blk(xbf, r, L, ch["c0"]),
                            ch["ag_p"][st])

        A, B = chains

        for win_ref, wout_ref in (
            (win0_ref, wout0_ref),
            (win1_ref, wout1_ref),
            (win2_ref, wout2_ref),
        ):
            h_act = jnp.maximum(
                jnp.dot(xbf[:, :], win_ref[:, :].astype(jnp.bfloat16),
                        preferred_element_type=jnp.float32),
                0.0,
            )
            partial[:, :] = jnp.dot(
                h_act.astype(jnp.bfloat16),
                wout_ref[:, :].astype(jnp.bfloat16),
                preferred_element_type=jnp.float32,
            )

            cast_send(A, 0)
            cast_send(B, 0)
            ra = rs_start(A, 0)
            rb = rs_start(B, 0)
            ra.wait()
            rs_finish(A, 0)
            cast_send(A, 1)
            ra = rs_start(A, 1)
            rb.wait()
            rs_finish(B, 0)
            cast_send(B, 1)
            rb = rs_start(B, 1)
            ra.wait()
            rs_finish(A, 1)
            cast_send(A, 2)
            ra = rs_start(A, 2)
            rb.wait()
            rs_finish(B, 1)
            cast_send(B, 2)
            rb = rs_start(B, 2)
            ra.wait()
            rs_finish(A, 2)
            ga = ag_start(A, 0)
            rb.wait()
            rs_finish(B, 2)
            gb = ag_start(B, 0)
            ga.wait()
            ga = ag_start(A, 1)
            gb.wait()
            gb = ag_start(B, 1)
            ga.wait()
            ga = ag_start(A, 2)
            gb.wait()
            gb = ag_start(B, 2)
            ga.wait()
            gb.wait()

        out_ref[:, :] = xbf[:, :].astype(jnp.float32)

    return pl.pallas_call(
        body,
        out_shape=jax.ShapeDtypeStruct((M, d), jnp.float32),
        in_specs=[pl.BlockSpec(memory_space=pltpu.VMEM)] * 7,
        out_specs=pl.BlockSpec(memory_space=pltpu.VMEM),
        scratch_shapes=[
            pltpu.VMEM((M, d), jnp.bfloat16),
            pltpu.VMEM((M, d), jnp.float32),
            pltpu.VMEM((M, d), jnp.bfloat16),
            pltpu.VMEM((2, 3, 4 * m_per, cw), jnp.bfloat16),
            pltpu.SemaphoreType.DMA((n_ex,)),
            pltpu.SemaphoreType.DMA((n_ex,)),
        ],
        compiler_params=pltpu.CompilerParams(collective_id=0),
    )(x, Win0, Wout0, Win1, Wout1, Win2, Wout2)


# device time: 88173 ns/iter; 2.8054x vs baseline; 1.0839x over previous
import jax
import jax.numpy as jnp
from jax import lax
from jax.experimental import pallas as pl
from jax.experimental.pallas import tpu as pltpu

N_DEV = 8


def kernel(x, Win0, Wout0, Win1, Wout1, Win2, Wout2):
    m_per, d = x.shape
    M = N_DEV * m_per
    cw = d // 2
    n_ex = 10 + 3 * 10

    def body(x_ref, win0_ref, wout0_ref, win1_ref, wout1_ref, win2_ref,
             wout2_ref, out_ref, xbf, partial, pbf, rs_recv,
             send_sems, recv_sems):
        my = lax.axis_index("i")
        p1 = jnp.bitwise_xor(my, 1)
        p2 = jnp.bitwise_xor(my, 3)
        p3 = jnp.bitwise_xor(my, 4)
        p7 = jnp.bitwise_xor(my, 7)
        b0 = jnp.bitwise_and(my, 1)
        b1 = jnp.bitwise_and(my >> 1, 1)
        b2 = jnp.bitwise_and(my >> 2, 1)

        barrier_sem = pltpu.get_barrier_semaphore()
        for p in (p1, p2, p3):
            pl.semaphore_signal(
                barrier_sem, inc=1,
                device_id=(p,), device_id_type=pl.DeviceIdType.MESH,
            )
        pl.semaphore_wait(barrier_sem, 3)

        ex = [0]

        def start_ex(src, dst, partner):
            k = ex[0]
            ex[0] += 1
            rdma = pltpu.make_async_remote_copy(
                src_ref=src,
                dst_ref=dst,
                send_sem=send_sems.at[k],
                recv_sem=recv_sems.at[k],
                device_id=(partner,),
                device_id_type=pl.DeviceIdType.MESH,
            )
            rdma.start()
            return rdma

        def xblk(r, L, c0):
            return xbf.at[pl.ds(r, L), pl.ds(c0, cw)]

        xbf[pl.ds(my * m_per, m_per), :] = x_ref[:, :].astype(jnp.bfloat16)
        a0 = my * m_per
        a1 = (my - b0) * m_per
        a2 = (my - jnp.bitwise_and(my, 3)) * m_per
        c3 = p2 * m_per
        ag_a = start_ex(xblk(a0, m_per, 0), xblk(a0, m_per, 0), p1)
        ag_b = start_ex(xblk(a0, m_per, cw), xblk(a0, m_per, cw), p2)
        ag_a.wait()
        ag_a = start_ex(xblk(a1, 2 * m_per, 0), xblk(a1, 2 * m_per, 0), p2)
        ag_b.wait()
        b2a = start_ex(xblk(a0, m_per, cw), xblk(a0, m_per, cw), p3)
        b2b = start_ex(xblk(c3, m_per, cw), xblk(c3, m_per, cw), p3)
        ag_a.wait()
        ag_a = start_ex(xblk(a2, 4 * m_per, 0), xblk(a2, 4 * m_per, 0), p3)
        b2a.wait()
        b2b.wait()
        b3 = [
            start_ex(xblk(c * m_per, m_per, cw), xblk(c * m_per, m_per, cw), p1)
            for c in (my, p2, p3, p7)
        ]
        ag_a.wait()
        for r in b3:
            r.wait()

        HA = b2 * (4 * m_per)
        QA = HA + b1 * (2 * m_per)
        HB = b1 * (4 * m_per)
        QB = HB + b2 * (2 * m_per)
        chains = (
            dict(c0=0, H=HA, Q=QA,
                 S=((1 - b2) * (4 * m_per), HA + (1 - b1) * (2 * m_per)),
                 rs_p=(p3, p2), ag_p=(p2, p3), idx=0),
            dict(c0=cw, H=HB, Q=QB,
                 S=((1 - b1) * (4 * m_per), HB + (1 - b2) * (2 * m_per)),
                 rs_p=(p2, p3), ag_p=(p3, p2), idx=1),
        )
        sizes = (4 * m_per, 2 * m_per)

        def cast_send(ch, r, L):
            c0 = ch["c0"]
            pbf[pl.ds(r, L), pl.ds(c0, cw)] = (
                partial[pl.ds(r, L), pl.ds(c0, cw)].astype(jnp.bfloat16))

        def rs_start(ch, st):
            r, L = ch["S"][st], sizes[st]
            return start_ex(
                pbf.at[pl.ds(r, L), pl.ds(ch["c0"], cw)],
                rs_recv.at[ch["idx"], st, pl.ds(0, L), :],
                ch["rs_p"][st],
            )

        def rs_finish(ch, st):
            k = (ch["H"], ch["Q"])[st]
            L, c0 = sizes[st], ch["c0"]
            partial[pl.ds(k, L), pl.ds(c0, cw)] = (
                partial[pl.ds(k, L), pl.ds(c0, cw)]
                + rs_recv[ch["idx"], st, 0:L, :].astype(jnp.float32)
            )

        def x_start(ch):
            cast_send(ch, ch["Q"], 2 * m_per)
            return start_ex(
                pbf.at[pl.ds(ch["Q"], 2 * m_per), pl.ds(ch["c0"], cw)],
                rs_recv.at[ch["idx"], 2, pl.ds(0, 2 * m_per), :],
                p1,
            )

        def x_finish(ch):
            Q, c0 = ch["Q"], ch["c0"]
            xbf[pl.ds(Q, 2 * m_per), pl.ds(c0, cw)] = (
                partial[pl.ds(Q, 2 * m_per), pl.ds(c0, cw)]
                + rs_recv[ch["idx"], 2, 0:2 * m_per, :].astype(jnp.float32)
            ).astype(jnp.bfloat16)

        def ag_start(ch, st):
            r = (ch["Q"], ch["H"])[st]
            L = (2 * m_per, 4 * m_per)[st]
            return start_ex(xblk(r, L, ch["c0"]), xblk(r, L, ch["c0"]),
                            ch["ag_p"][st])

        A, B = chains

        for win_ref, wout_ref in (
            (win0_ref, wout0_ref),
            (win1_ref, wout1_ref),
            (win2_ref, wout2_ref),
        ):
            h_act = jnp.maximum(
                jnp.dot(xbf[:, :], win_ref[:, :].astype(jnp.bfloat16),
                        preferred_element_type=jnp.float32),
                0.0,
            )
            partial[:, :] = jnp.dot(
                h_act.astype(jnp.bfloat16),
                wout_ref[:, :].astype(jnp.bfloat16),
                preferred_element_type=jnp.float32,
            )

            cast_send(A, A["S"][0], sizes[0])
            cast_send(B, B["S"][0], sizes[0])
            ra = rs_start(A, 0)
            rb = rs_start(B, 0)
            ra.wait()
            rs_finish(A, 0)
            cast_send(A, A["S"][1], sizes[1])
            ra = rs_start(A, 1)
            rb.wait()
            rs_finish(B, 0)
            cast_send(B, B["S"][1], sizes[1])
            rb = rs_start(B, 1)
            ra.wait()
            rs_finish(A, 1)
            ra = x_start(A)
            rb.wait()
            rs_finish(B, 1)
            rb = x_start(B)
            ra.wait()
            x_finish(A)
            ga = ag_start(A, 0)
            rb.wait()
            x_finish(B)
            gb = ag_start(B, 0)
            ga.wait()
            ga = ag_start(A, 1)
            gb.wait()
            gb = ag_start(B, 1)
            ga.wait()
            gb.wait()

        out_ref[:, :] = xbf[:, :].astype(jnp.float32)

    return pl.pallas_call(
        body,
        out_shape=jax.ShapeDtypeStruct((M, d), jnp.float32),
        in_specs=[pl.BlockSpec(memory_space=pltpu.VMEM)] * 7,
        out_specs=pl.BlockSpec(memory_space=pltpu.VMEM),
        scratch_shapes=[
            pltpu.VMEM((M, d), jnp.bfloat16),
            pltpu.VMEM((M, d), jnp.float32),
            pltpu.VMEM((M, d), jnp.bfloat16),
            pltpu.VMEM((2, 3, 4 * m_per, cw), jnp.bfloat16),
            pltpu.SemaphoreType.DMA((n_ex,)),
            pltpu.SemaphoreType.DMA((n_ex,)),
        ],
        compiler_params=pltpu.CompilerParams(collective_id=0),
    )(x, Win0, Wout0, Win1, Wout1, Win2, Wout2)
